# baseline (speedup 1.0000x reference)
import functools

import jax
import jax.numpy as jnp
from jax.experimental import pallas as pl
from jax.experimental.pallas import tpu as pltpu

_N_SYS = 2048
_EPS = 1e-6
_BSYS = 16


def _seg_kernel(s_boff_ref, p_boff_ref,
                s_slot_ref, s_m2cx_ref, s_m2cy_ref, s_cc_ref, s_th_ref,
                s_cx_ref, s_cy_ref,
                p_slot_ref, p_m2cx_ref, p_m2cy_ref, p_cc_ref, p_b0_ref,
                p_b1_ref, p_cx_ref, p_cy_ref,
                g2_ref, gx_ref, gy_ref, out_ref,
                acca_ref, accbx_ref, accby_ref, *, rr, cc):
    s = pl.program_id(0)
    g2 = g2_ref[...]
    gxp = gx_ref[...]
    gyp = gy_ref[...]

    zero = jnp.zeros((rr, cc), jnp.float32)
    for j in range(_BSYS):
        acca_ref[j] = zero
        accbx_ref[j] = zero
        accby_ref[j] = zero

    def sis_body(i, carry):
        slot = s_slot_ref[i]
        u = g2 + s_cc_ref[i]
        u = u + s_m2cx_ref[i] * gxp
        u = u + s_m2cy_ref[i] * gyp
        coef = s_th_ref[i] * jax.lax.rsqrt(u)
        acca_ref[slot] += coef
        accbx_ref[slot] += s_cx_ref[i] * coef
        accby_ref[slot] += s_cy_ref[i] * coef
        return carry

    def pemd_body(i, carry):
        slot = p_slot_ref[i]
        u = g2 + p_cc_ref[i]
        u = u + p_m2cx_ref[i] * gxp
        u = u + p_m2cy_ref[i] * gyp
        coef = jnp.exp(p_b0_ref[i] + p_b1_ref[i] * jnp.log(u))
        coef = coef * jax.lax.rsqrt(u)
        acca_ref[slot] += coef
        accbx_ref[slot] += p_cx_ref[i] * coef
        accby_ref[slot] += p_cy_ref[i] * coef
        return carry

    jax.lax.fori_loop(s_boff_ref[s], s_boff_ref[s + 1], sis_body, 0)
    jax.lax.fori_loop(p_boff_ref[s], p_boff_ref[s + 1], pemd_body, 0)

    for j in range(_BSYS):
        na = 1.0 - acca_ref[j]
        out_ref[j, 0] = gxp * na + accbx_ref[j]
        out_ref[j, 1] = gyp * na + accby_ref[j]


def _roundrobin(idx):
    n = idx.shape[0]
    order0 = jnp.argsort(idx)
    sid0 = idx[order0]
    counts = jnp.bincount(idx, length=_N_SYS)
    off = jnp.concatenate(
        [jnp.zeros((1,), jnp.int32),
         jnp.cumsum(counts).astype(jnp.int32)])
    rank = jnp.arange(n, dtype=jnp.int32) - off[sid0]
    block = sid0 // _BSYS
    slot = sid0 % _BSYS
    key = (block * 8192 + rank) * _BSYS + slot
    order1 = jnp.argsort(key)
    perm = order0[order1]
    bcounts = jnp.bincount(block, length=_N_SYS // _BSYS)
    block_off = jnp.concatenate(
        [jnp.zeros((1,), jnp.int32),
         jnp.cumsum(bcounts).astype(jnp.int32)])
    return perm, slot[order1].astype(jnp.int32), block_off


@jax.jit
def kernel(lens_grid, sis_params, pemd_params, sis_idx, pemd_idx):
    hh, ww, _ = lens_grid.shape
    rr = hh * ww // 128
    gx = lens_grid[:, :, 0].reshape(rr, 128)
    gy = lens_grid[:, :, 1].reshape(rr, 128)
    g2 = gx * gx + gy * gy + _EPS

    s_perm, s_slot, s_boff = _roundrobin(sis_idx)
    sp = sis_params[s_perm]
    s_th, s_cx, s_cy = sp[:, 0], sp[:, 1], sp[:, 2]
    s_m2cx = -2.0 * s_cx
    s_m2cy = -2.0 * s_cy
    s_cc = s_cx * s_cx + s_cy * s_cy

    p_perm, p_slot, p_boff = _roundrobin(pemd_idx)
    pp = pemd_params[p_perm]
    th, gam, p_cx, p_cy = pp[:, 0], pp[:, 1], pp[:, 2], pp[:, 3]
    p_b0 = (gam - 1.0) * jnp.log(th)
    p_b1 = 0.5 * (2.0 - gam)
    p_m2cx = -2.0 * p_cx
    p_m2cy = -2.0 * p_cy
    p_cc = p_cx * p_cx + p_cy * p_cy

    out = pl.pallas_call(
        functools.partial(_seg_kernel, rr=rr, cc=128),
        grid=(_N_SYS // _BSYS,),
        in_specs=[pl.BlockSpec(memory_space=pltpu.SMEM)] * 17 + [
            pl.BlockSpec((rr, 128), lambda s: (0, 0)),
            pl.BlockSpec((rr, 128), lambda s: (0, 0)),
            pl.BlockSpec((rr, 128), lambda s: (0, 0)),
        ],
        out_specs=pl.BlockSpec((_BSYS, 2, rr, 128), lambda s: (s, 0, 0, 0)),
        out_shape=jax.ShapeDtypeStruct((_N_SYS, 2, rr, 128), jnp.float32),
        scratch_shapes=[
            pltpu.VMEM((_BSYS, rr, 128), jnp.float32),
            pltpu.VMEM((_BSYS, rr, 128), jnp.float32),
            pltpu.VMEM((_BSYS, rr, 128), jnp.float32),
        ],
    )(s_boff, p_boff,
      s_slot, s_m2cx, s_m2cy, s_cc, s_th, s_cx, s_cy,
      p_slot, p_m2cx, p_m2cy, p_cc, p_b0, p_b1, p_cx, p_cy,
      g2, gx, gy)
    return out.reshape(_N_SYS, 2, hh, ww).transpose(0, 2, 3, 1)


# speedup vs baseline: 1.2252x; 1.2252x over previous
import functools

import jax
import jax.numpy as jnp
from jax.experimental import pallas as pl
from jax.experimental.pallas import tpu as pltpu

_N_SYS = 2048
_EPS = 1e-6


def _seg_kernel(off_ref, m2cx_ref, m2cy_ref, cc_ref, th_ref, b0_ref,
                b1_ref, cx_ref, cy_ref, g2_ref, gx_ref, gy_ref, out_ref,
                *, rr, cc, bsys):
    s = pl.program_id(0)
    g2 = g2_ref[...]
    gxp = gx_ref[...]
    gyp = gy_ref[...]

    def sis_body(i, carry):
        a, bx, by = carry
        u = g2 + cc_ref[i]
        u = u + m2cx_ref[i] * gxp
        u = u + m2cy_ref[i] * gyp
        coef = th_ref[i] * jax.lax.rsqrt(u)
        return a + coef, bx + cx_ref[i] * coef, by + cy_ref[i] * coef

    def pemd_body(i, carry):
        a, bx, by = carry
        u = g2 + cc_ref[i]
        u = u + m2cx_ref[i] * gxp
        u = u + m2cy_ref[i] * gyp
        coef = jnp.exp2(b0_ref[i] + b1_ref[i] * jnp.log2(u))
        coef = coef * jax.lax.rsqrt(u)
        return a + coef, bx + cx_ref[i] * coef, by + cy_ref[i] * coef

    zero = jnp.zeros((rr, cc), jnp.float32)
    for j in range(bsys):
        sysid = s * bsys + j
        lo = off_ref[2 * sysid]
        mid = off_ref[2 * sysid + 1]
        hi = off_ref[2 * sysid + 2]
        carry = jax.lax.fori_loop(lo, mid, sis_body, (zero, zero, zero))
        a, bx, by = jax.lax.fori_loop(mid, hi, pemd_body, carry)
        na = 1.0 - a
        out_ref[j, 0] = gxp * na + bx
        out_ref[j, 1] = gyp * na + by


@jax.jit
def kernel(lens_grid, sis_params, pemd_params, sis_idx, pemd_idx):
    hh, ww, _ = lens_grid.shape
    rr = hh * ww // 128
    gx = lens_grid[:, :, 0].reshape(rr, 128)
    gy = lens_grid[:, :, 1].reshape(rr, 128)
    g2 = gx * gx + gy * gy + _EPS

    th = jnp.concatenate([sis_params[:, 0], pemd_params[:, 0]])
    gam = jnp.concatenate([jnp.full(sis_params.shape[:1], 2.0),
                           pemd_params[:, 1]])
    cx = jnp.concatenate([sis_params[:, 1], pemd_params[:, 2]])
    cy = jnp.concatenate([sis_params[:, 2], pemd_params[:, 3]])
    typ = jnp.concatenate([jnp.zeros(sis_idx.shape, jnp.int32),
                           jnp.ones(pemd_idx.shape, jnp.int32)])
    idx2 = 2 * jnp.concatenate([sis_idx, pemd_idx]) + typ

    order = jnp.argsort(idx2)
    th = th[order]
    gam = gam[order]
    cx = cx[order]
    cy = cy[order]

    b0 = (gam - 1.0) * jnp.log2(th)
    b1 = 0.5 * (2.0 - gam)
    m2cx = -2.0 * cx
    m2cy = -2.0 * cy
    ccs = cx * cx + cy * cy
    counts = jnp.bincount(idx2, length=2 * _N_SYS)
    off = jnp.concatenate(
        [jnp.zeros((1,), jnp.int32),
         jnp.cumsum(counts).astype(jnp.int32)])

    bsys = 32
    out = pl.pallas_call(
        functools.partial(_seg_kernel, rr=rr, cc=128, bsys=bsys),
        grid=(_N_SYS // bsys,),
        in_specs=[pl.BlockSpec(memory_space=pltpu.SMEM)] * 9 + [
            pl.BlockSpec((rr, 128), lambda s: (0, 0)),
            pl.BlockSpec((rr, 128), lambda s: (0, 0)),
            pl.BlockSpec((rr, 128), lambda s: (0, 0)),
        ],
        out_specs=pl.BlockSpec((bsys, 2, rr, 128), lambda s: (s, 0, 0, 0)),
        out_shape=jax.ShapeDtypeStruct((_N_SYS, 2, rr, 128), jnp.float32),
    )(off, m2cx, m2cy, ccs, th, b0, b1, cx, cy, g2, gx, gy)
    return out.reshape(_N_SYS, 2, hh, ww).transpose(0, 2, 3, 1)


# speedup vs baseline: 1.2299x; 1.0038x over previous
import functools

import jax
import jax.numpy as jnp
from jax.experimental import pallas as pl
from jax.experimental.pallas import tpu as pltpu

_N_SYS = 2048
_EPS = 1e-6


def _seg_kernel(off_ref, m2cx_ref, m2cy_ref, cc_ref, th_ref, b0_ref,
                b1_ref, cx_ref, cy_ref, g2_ref, gx_ref, gy_ref, out_ref,
                *, rr, cc, bsys):
    s = pl.program_id(0)
    g2 = g2_ref[...]
    gxp = gx_ref[...]
    gyp = gy_ref[...]

    def sis_body(i, carry):
        a, bx, by = carry
        u = g2 + cc_ref[i]
        u = u + m2cx_ref[i] * gxp
        u = u + m2cy_ref[i] * gyp
        coef = th_ref[i] * jax.lax.rsqrt(u)
        return a + coef, bx + cx_ref[i] * coef, by + cy_ref[i] * coef

    def pemd_body(i, carry):
        a, bx, by = carry
        u = g2 + cc_ref[i]
        u = u + m2cx_ref[i] * gxp
        u = u + m2cy_ref[i] * gyp
        coef = jnp.exp2(b0_ref[i] + b1_ref[i] * jnp.log2(u))
        coef = coef * jax.lax.rsqrt(u)
        return a + coef, bx + cx_ref[i] * coef, by + cy_ref[i] * coef

    zero = jnp.zeros((rr, cc), jnp.float32)
    for j in range(bsys):
        sysid = s * bsys + j
        lo = off_ref[2 * sysid]
        mid = off_ref[2 * sysid + 1]
        hi = off_ref[2 * sysid + 2]
        carry = jax.lax.fori_loop(lo, mid, sis_body, (zero, zero, zero))
        a, bx, by = jax.lax.fori_loop(mid, hi, pemd_body, carry)
        na = 1.0 - a
        out_ref[j, 0] = gxp * na + bx
        out_ref[j, 1] = gyp * na + by


@jax.jit
def kernel(lens_grid, sis_params, pemd_params, sis_idx, pemd_idx):
    hh, ww, _ = lens_grid.shape
    rr = hh * ww // 128
    gx = lens_grid[:, :, 0].reshape(rr, 128)
    gy = lens_grid[:, :, 1].reshape(rr, 128)
    g2 = gx * gx + gy * gy + _EPS

    th = jnp.concatenate([sis_params[:, 0], pemd_params[:, 0]])
    gam = jnp.concatenate([jnp.full(sis_params.shape[:1], 2.0),
                           pemd_params[:, 1]])
    cx = jnp.concatenate([sis_params[:, 1], pemd_params[:, 2]])
    cy = jnp.concatenate([sis_params[:, 2], pemd_params[:, 3]])
    typ = jnp.concatenate([jnp.zeros(sis_idx.shape, jnp.int32),
                           jnp.ones(pemd_idx.shape, jnp.int32)])
    idx2 = 2 * jnp.concatenate([sis_idx, pemd_idx]) + typ

    order = jnp.argsort(idx2)
    th = th[order]
    gam = gam[order]
    cx = cx[order]
    cy = cy[order]

    b0 = (gam - 1.0) * jnp.log2(th)
    b1 = 0.5 * (2.0 - gam)
    m2cx = -2.0 * cx
    m2cy = -2.0 * cy
    ccs = cx * cx + cy * cy
    counts = jnp.bincount(idx2, length=2 * _N_SYS)
    off = jnp.concatenate(
        [jnp.zeros((1,), jnp.int32),
         jnp.cumsum(counts).astype(jnp.int32)])

    bsys = 64
    out = pl.pallas_call(
        functools.partial(_seg_kernel, rr=rr, cc=128, bsys=bsys),
        grid=(_N_SYS // bsys,),
        in_specs=[pl.BlockSpec(memory_space=pltpu.SMEM)] * 9 + [
            pl.BlockSpec((rr, 128), lambda s: (0, 0)),
            pl.BlockSpec((rr, 128), lambda s: (0, 0)),
            pl.BlockSpec((rr, 128), lambda s: (0, 0)),
        ],
        out_specs=pl.BlockSpec((bsys, 2, rr, 128), lambda s: (s, 0, 0, 0)),
        out_shape=jax.ShapeDtypeStruct((_N_SYS, 2, rr, 128), jnp.float32),
    )(off, m2cx, m2cy, ccs, th, b0, b1, cx, cy, g2, gx, gy)
    return out.reshape(_N_SYS, 2, hh, ww).transpose(0, 2, 3, 1)


# speedup vs baseline: 1.2540x; 1.0196x over previous
import functools

import jax
import jax.numpy as jnp
from jax.experimental import pallas as pl
from jax.experimental.pallas import tpu as pltpu

_N_SYS = 2048
_EPS = 1e-6


def _seg_kernel(off_ref, ord_ref, m2cx_ref, m2cy_ref, cc_ref, th_ref,
                b0_ref, b1_ref, cx_ref, cy_ref, g2_ref, gx_ref, gy_ref,
                out_ref, *, rr, cc, bsys):
    s = pl.program_id(0)
    g2 = g2_ref[...]
    gxp = gx_ref[...]
    gyp = gy_ref[...]

    def sis_body(i, carry):
        a, bx, by = carry
        k = ord_ref[i]
        u = g2 + cc_ref[k]
        u = u + m2cx_ref[k] * gxp
        u = u + m2cy_ref[k] * gyp
        coef = th_ref[k] * jax.lax.rsqrt(u)
        return a + coef, bx + cx_ref[k] * coef, by + cy_ref[k] * coef

    def pemd_body(i, carry):
        a, bx, by = carry
        k = ord_ref[i]
        u = g2 + cc_ref[k]
        u = u + m2cx_ref[k] * gxp
        u = u + m2cy_ref[k] * gyp
        coef = jnp.exp2(b0_ref[k] + b1_ref[k] * jnp.log2(u))
        coef = coef * jax.lax.rsqrt(u)
        return a + coef, bx + cx_ref[k] * coef, by + cy_ref[k] * coef

    zero = jnp.zeros((rr, cc), jnp.float32)
    for j in range(bsys):
        sysid = s * bsys + j
        lo = off_ref[2 * sysid]
        mid = off_ref[2 * sysid + 1]
        hi = off_ref[2 * sysid + 2]
        carry = jax.lax.fori_loop(lo, mid, sis_body, (zero, zero, zero))
        a, bx, by = jax.lax.fori_loop(mid, hi, pemd_body, carry)
        na = 1.0 - a
        out_ref[j, 0] = gxp * na + bx
        out_ref[j, 1] = gyp * na + by


@jax.jit
def kernel(lens_grid, sis_params, pemd_params, sis_idx, pemd_idx):
    hh, ww, _ = lens_grid.shape
    rr = hh * ww // 128
    gx = lens_grid[:, :, 0].reshape(rr, 128)
    gy = lens_grid[:, :, 1].reshape(rr, 128)
    g2 = gx * gx + gy * gy + _EPS

    th = jnp.concatenate([sis_params[:, 0], pemd_params[:, 0]])
    gam = jnp.concatenate([jnp.full(sis_params.shape[:1], 2.0),
                           pemd_params[:, 1]])
    cx = jnp.concatenate([sis_params[:, 1], pemd_params[:, 2]])
    cy = jnp.concatenate([sis_params[:, 2], pemd_params[:, 3]])
    typ = jnp.concatenate([jnp.zeros(sis_idx.shape, jnp.int32),
                           jnp.ones(pemd_idx.shape, jnp.int32)])
    idx2 = 2 * jnp.concatenate([sis_idx, pemd_idx]) + typ

    order = jnp.argsort(idx2).astype(jnp.int32)

    b0 = (gam - 1.0) * jnp.log2(th)
    b1 = 0.5 * (2.0 - gam)
    m2cx = -2.0 * cx
    m2cy = -2.0 * cy
    ccs = cx * cx + cy * cy
    counts = jnp.bincount(idx2, length=2 * _N_SYS)
    off = jnp.concatenate(
        [jnp.zeros((1,), jnp.int32),
         jnp.cumsum(counts).astype(jnp.int32)])

    bsys = 64
    out = pl.pallas_call(
        functools.partial(_seg_kernel, rr=rr, cc=128, bsys=bsys),
        grid=(_N_SYS // bsys,),
        in_specs=[pl.BlockSpec(memory_space=pltpu.SMEM)] * 10 + [
            pl.BlockSpec((rr, 128), lambda s: (0, 0)),
            pl.BlockSpec((rr, 128), lambda s: (0, 0)),
            pl.BlockSpec((rr, 128), lambda s: (0, 0)),
        ],
        out_specs=pl.BlockSpec((bsys, 2, rr, 128), lambda s: (s, 0, 0, 0)),
        out_shape=jax.ShapeDtypeStruct((_N_SYS, 2, rr, 128), jnp.float32),
    )(off, order, m2cx, m2cy, ccs, th, b0, b1, cx, cy, g2, gx, gy)
    return out.reshape(_N_SYS, 2, hh, ww).transpose(0, 2, 3, 1)
